# baseline (speedup 1.0000x reference)
import functools

import jax
import jax.numpy as jnp
from jax import lax
from jax.experimental import pallas as pl
from jax.experimental.pallas import tpu as pltpu
from jax.experimental.pallas import tpu_sc as plsc

B, H, D = 16384, 50, 16
N = B * H
NC, NS = 2, 16
NW = NC * NS
ROWS_PER_W = N // NW
K = 8
C = K * 128
STEPS = ROWS_PER_W // C
IDX_ROWS_PER_W = ROWS_PER_W // 128

_mesh = plsc.VectorSubcoreMesh(
    core_axis_name="c", subcore_axis_name="s", num_cores=NC, num_subcores=NS
)


@functools.partial(
    pl.kernel,
    out_type=jax.ShapeDtypeStruct((N, D), jnp.float32),
    mesh=_mesh,
    compiler_params=pltpu.CompilerParams(use_tc_tiling_on_sc=False),
    scratch_types=[
        pltpu.VMEM((K, 128), jnp.int32),
        pltpu.VMEM((C, D), jnp.float32),
        pltpu.SemaphoreType.DMA,
        pltpu.SemaphoreType.DMA,
    ],
)
def _encode(item_hbm, ent_hbm, idx_hbm, out_hbm, idx_v, rows_v, sem_a, sem_b):
    wid = lax.axis_index("s") * NC + lax.axis_index("c")
    idx_row0 = wid * IDX_ROWS_PER_W
    out_row0 = wid * ROWS_PER_W

    @pl.loop(0, STEPS)
    def _step(s):
        pltpu.sync_copy(idx_hbm.at[pl.ds(idx_row0 + s * K, K)], idx_v)
        cps = [
            pltpu.async_copy(
                item_hbm.at[idx_v.at[j]], rows_v.at[pl.ds(j * 128, 128)], sem_a
            )
            for j in range(K)
        ]
        for cp in cps:
            cp.wait()
        cps = [
            pltpu.async_copy(
                ent_hbm.at[idx_v.at[j]],
                rows_v.at[pl.ds(j * 128, 128)],
                sem_b,
                add=True,
            )
            for j in range(K)
        ]
        for cp in cps:
            cp.wait()
        pltpu.sync_copy(rows_v, out_hbm.at[pl.ds(out_row0 + s * C, C)])


def kernel(batch_data, item_table, ent_table):
    idx = batch_data.reshape(N // 128, 128).astype(jnp.int32)
    out = _encode(item_table, ent_table, idx)
    return out.reshape(B, H, D)
